# baseline (speedup 1.0000x reference)
import functools

import jax
import jax.numpy as jnp
from jax import lax
from jax.experimental import pallas as pl
from jax.experimental.pallas import tpu as pltpu
from jax.experimental.pallas import tpu_sc as plsc

ADMP = 8
B, C, H, W = 8, 384, 32, 32
HW = H * W
N = ADMP * ADMP
KH = H // ADMP
NCHUNK = 16
CW = HW // NCHUNK

_BIG_I = 2**30
_NEG = -3.0e38


def _main_body(xA_ref, xB_ref, cmv_ref, cmi_ref, kp_ref):
    A = xA_ref[0]
    Bf = xB_ref[0]

    resp = jnp.sum(A, axis=0, keepdims=True)
    p = jax.lax.broadcasted_iota(jnp.int32, (N, HW), 1)
    wofp = (p // W // KH) * ADMP + (p % W) // KH
    wrow = jax.lax.broadcasted_iota(jnp.int32, (N, HW), 0)
    inwin = wofp == wrow
    mresp = jnp.where(inwin, jnp.broadcast_to(resp, (N, HW)), _NEG)
    wmax = jnp.max(mresp, axis=1, keepdims=True)
    cand = jnp.where(inwin & (mresp == wmax), p, _BIG_I)
    kp = jnp.min(cand, axis=1)

    prow = jax.lax.broadcasted_iota(jnp.int32, (HW, N), 0)
    onehot = (prow == kp[None, :]).astype(jnp.float32)
    desc = jax.lax.dot(A, onehot, preferred_element_type=jnp.float32)

    a2 = jnp.sum(A * A, axis=0, keepdims=True)
    kpmask = p == kp[:, None]
    d2 = jnp.sum(jnp.where(kpmask, jnp.broadcast_to(a2, (N, HW)), 0.0),
                 axis=1)[:, None]
    f2 = jnp.sum(Bf * Bf, axis=0)[None, :]
    cross = jax.lax.dot_general(desc, Bf, (((0,), (0,)), ((), ())),
                                preferred_element_type=jnp.float32)
    dist = d2 + f2 - 2.0 * cross

    mins = []
    idxs = []
    for c in range(NCHUNK):
        ch = dist[:, c * CW:(c + 1) * CW]
        pch = p[:, c * CW:(c + 1) * CW]
        m = jnp.min(ch, axis=1, keepdims=True)
        gi = jnp.min(jnp.where(ch == m, pch, _BIG_I), axis=1, keepdims=True)
        mins.append(m)
        idxs.append(gi)
    cmv_ref[0] = lax.transpose(jnp.concatenate(mins, axis=1), (1, 0))
    cmi_ref[0] = lax.transpose(jnp.concatenate(idxs, axis=1), (1, 0))
    kp_ref[0, 0] = kp


def _sc_merge_body(cmv_hbm, cmi_hbm, kp_hbm, drow_hbm, dcol_hbm,
                   cmv_v, cmi_v, kp_v, drow_v, dcol_v):
    info = plsc.get_sparse_core_info()
    nc = info.num_cores
    wid = lax.axis_index("s") * nc + lax.axis_index("c")
    b = wid // 4
    k0 = (wid % 4) * 16

    pltpu.sync_copy(cmv_hbm.at[b], cmv_v)
    pltpu.sync_copy(cmi_hbm.at[b], cmi_v)
    pltpu.sync_copy(kp_hbm.at[b, pl.ds(k0, 16)], kp_v)

    best = jnp.zeros((16,), jnp.float32) + 3.0e38
    minv = jnp.zeros((16,), jnp.int32)
    for c in range(NCHUNK):
        vals = cmv_v[c, pl.ds(k0, 16)]
        ids = cmi_v[c, pl.ds(k0, 16)]
        better = vals < best
        minv = jnp.where(better, ids, minv)
        best = jnp.where(better, vals, best)

    kpv = kp_v[...]
    row_A = kpv >> 5
    col_A = kpv & 31
    row_B = minv >> 5
    col_B = minv & 31
    drow_v[...] = (row_B - row_A).astype(jnp.float32)
    dcol_v[...] = (col_A - col_B).astype(jnp.float32)
    pltpu.sync_copy(drow_v, drow_hbm.at[b, pl.ds(k0, 16)])
    pltpu.sync_copy(dcol_v, dcol_hbm.at[b, pl.ds(k0, 16)])


_sc_merge = functools.partial(
    pl.kernel,
    mesh=plsc.VectorSubcoreMesh(core_axis_name="c", subcore_axis_name="s"),
    out_type=[jax.ShapeDtypeStruct((B, N), jnp.float32),
              jax.ShapeDtypeStruct((B, N), jnp.float32)],
    scratch_types=[pltpu.VMEM((NCHUNK, N), jnp.float32),
                   pltpu.VMEM((NCHUNK, N), jnp.int32),
                   pltpu.VMEM((16,), jnp.int32),
                   pltpu.VMEM((16,), jnp.float32),
                   pltpu.VMEM((16,), jnp.float32)],
)(_sc_merge_body)


def _mlp_body(drow_ref, dcol_ref, W1r_ref, b1r_ref, W2r_ref, b2r_ref,
              W1c_ref, b1c_ref, W2c_ref, b2c_ref, out_ref):
    drow = drow_ref[...]
    dcol = dcol_ref[...]
    hr = jnp.maximum(
        jax.lax.dot_general(drow, W1r_ref[...], (((1,), (1,)), ((), ())),
                            preferred_element_type=jnp.float32) + b1r_ref[...],
        0.0)
    w2r = W2r_ref[...].astype(jnp.bfloat16).astype(jnp.float32)
    orr = jnp.sum(hr.astype(jnp.bfloat16).astype(jnp.float32) * w2r,
                  axis=1, keepdims=True) + b2r_ref[0, 0]
    hc = jnp.maximum(
        jax.lax.dot_general(dcol, W1c_ref[...], (((1,), (1,)), ((), ())),
                            preferred_element_type=jnp.float32) + b1c_ref[...],
        0.0)
    w2c = W2c_ref[...].astype(jnp.bfloat16).astype(jnp.float32)
    occ = jnp.sum(hc.astype(jnp.bfloat16).astype(jnp.float32) * w2c,
                  axis=1, keepdims=True) + b2c_ref[0, 0]
    out_ref[...] = jnp.concatenate([orr, occ], axis=1)


def kernel(xA, xB, W1r, b1r, W2r, b2r, W1c, b1c, W2c, b2c):
    xA3 = xA.reshape(B, C, HW)
    xB3 = xB.reshape(B, C, HW)

    cmv, cmi, kp3 = pl.pallas_call(
        _main_body,
        grid=(B,),
        in_specs=[
            pl.BlockSpec((1, C, HW), lambda b: (b, 0, 0)),
            pl.BlockSpec((1, C, HW), lambda b: (b, 0, 0)),
        ],
        out_specs=[
            pl.BlockSpec((1, NCHUNK, N), lambda b: (b, 0, 0)),
            pl.BlockSpec((1, NCHUNK, N), lambda b: (b, 0, 0)),
            pl.BlockSpec((1, 1, N), lambda b: (b, 0, 0)),
        ],
        out_shape=[
            jax.ShapeDtypeStruct((B, NCHUNK, N), jnp.float32),
            jax.ShapeDtypeStruct((B, NCHUNK, N), jnp.int32),
            jax.ShapeDtypeStruct((B, 1, N), jnp.int32),
        ],
        compiler_params=pltpu.CompilerParams(
            dimension_semantics=("parallel",)),
    )(xA3, xB3)

    drow, dcol = _sc_merge(cmv, cmi, kp3.reshape(B, N))

    full = lambda s: pl.BlockSpec(s, lambda i: (0,) * len(s))
    out = pl.pallas_call(
        _mlp_body,
        grid=(1,),
        in_specs=[
            full((B, N)), full((B, N)),
            full((N // 2, N)), full((1, N // 2)),
            full((1, N // 2)), pl.BlockSpec(memory_space=pltpu.SMEM),
            full((N // 2, N)), full((1, N // 2)),
            full((1, N // 2)), pl.BlockSpec(memory_space=pltpu.SMEM),
        ],
        out_specs=pl.BlockSpec((B, 2), lambda i: (0, 0)),
        out_shape=jax.ShapeDtypeStruct((B, 2), jnp.float32),
    )(drow, dcol, W1r, b1r.reshape(1, N // 2), W2r, b2r.reshape(1, 1),
      W1c, b1c.reshape(1, N // 2), W2c, b2c.reshape(1, 1))
    return out
